# Initial kernel scaffold; baseline (speedup 1.0000x reference)
import jax
import jax.numpy as jnp
from jax.experimental import pallas as pl


def kernel(indices, table):
    raise NotImplementedError("write your pallas kernel here")


# speedup vs baseline: 1.5766x; 1.5766x over previous
import functools

import jax
import jax.numpy as jnp
from jax import lax
from jax.experimental import pallas as pl
from jax.experimental.pallas import tpu as pltpu
from jax.experimental.pallas import tpu_sc as plsc


def _make_gather(B, V, D):
    info = plsc.get_sparse_core_info()
    NC, NS = info.num_cores, info.num_subcores
    NW = NC * NS
    assert B % NW == 0
    b_per_w = B // NW
    NCHUNK = 8
    assert b_per_w % NCHUNK == 0
    C = b_per_w // NCHUNK
    assert C % 8 == 0

    mesh = plsc.VectorSubcoreMesh(core_axis_name="c", subcore_axis_name="s")

    @functools.partial(
        pl.kernel,
        mesh=mesh,
        out_type=jax.ShapeDtypeStruct((B, D), jnp.float32),
        scratch_types=[
            pltpu.VMEM((2, C), jnp.int32),
            pltpu.VMEM((2, C, D), jnp.float32),
            pltpu.SemaphoreType.DMA,
            pltpu.SemaphoreType.DMA,
        ],
        compiler_params=pltpu.CompilerParams(use_tc_tiling_on_sc=False),
    )
    def k(idx_hbm, table_hbm, out_hbm, idx_v, rows_v, gsem, osem):
        wid = lax.axis_index("s") * NC + lax.axis_index("c")
        base = wid * b_per_w

        gathers = [None, None]
        stores = [None, None]
        for c in range(min(2, NCHUNK)):
            b = c % 2
            pltpu.sync_copy(idx_hbm.at[pl.ds(base + c * C, C)], idx_v.at[b])
            gathers[b] = pltpu.async_copy(
                table_hbm.at[idx_v.at[b]], rows_v.at[b], gsem
            )
        for c in range(NCHUNK):
            b = c % 2
            gathers[b].wait()
            stores[b] = pltpu.async_copy(
                rows_v.at[b], out_hbm.at[pl.ds(base + c * C, C)], osem
            )
            if c + 2 < NCHUNK:
                pltpu.sync_copy(
                    idx_hbm.at[pl.ds(base + (c + 2) * C, C)], idx_v.at[b]
                )
                stores[b].wait()
                gathers[b] = pltpu.async_copy(
                    table_hbm.at[idx_v.at[b]], rows_v.at[b], gsem
                )
        stores[(NCHUNK - 2) % 2].wait()
        stores[(NCHUNK - 1) % 2].wait()

    return k


def kernel(indices, table):
    Bf, L = indices.shape
    V, D = table.shape
    flat_idx = indices.reshape(Bf * L)
    gathered = _make_gather(Bf * L, V, D)(flat_idx, table)
    return gathered.reshape(Bf, L, D)
